# baseline (device time: 13741 ns/iter reference)
import jax
import jax.numpy as jnp
from jax import lax
from jax.experimental import pallas as pl
from jax.experimental.pallas import tpu as pltpu

N_DEV = 8


def kernel(x, w_mat):
    m_per, k = x.shape
    n = w_mat.shape[1]
    n_per = n // N_DEV

    def body(x_ref, w_ref, out_ref, comm_ref, send_sems, recv_sems):
        my = lax.axis_index("i")

        barrier_sem = pltpu.get_barrier_semaphore()
        for o in range(1, N_DEV):
            peer = lax.rem(my + o, N_DEV)
            pl.semaphore_signal(
                barrier_sem, inc=1,
                device_id=(peer,), device_id_type=pl.DeviceIdType.MESH,
            )
        pl.semaphore_wait(barrier_sem, N_DEV - 1)

        x_blk = x_ref[:, :]
        sends = []
        for o in range(1, N_DEV):
            j = lax.rem(my + o, N_DEV)
            comm_ref[o, :, :] = jnp.dot(
                x_blk,
                w_ref[:, pl.ds(j * n_per, n_per)],
                preferred_element_type=jnp.float32,
            )
            rdma = pltpu.make_async_remote_copy(
                src_ref=comm_ref.at[o],
                dst_ref=out_ref.at[pl.ds(my * m_per, m_per)],
                send_sem=send_sems.at[o],
                recv_sem=recv_sems.at[my],
                device_id=(j,),
                device_id_type=pl.DeviceIdType.MESH,
            )
            rdma.start()
            sends.append(rdma)

        out_ref[pl.ds(my * m_per, m_per), :] = jnp.dot(
            x_blk,
            w_ref[:, pl.ds(my * n_per, n_per)],
            preferred_element_type=jnp.float32,
        )

        for o in range(1, N_DEV):
            s = lax.rem(my + N_DEV - o, N_DEV)
            recv = pltpu.make_async_remote_copy(
                src_ref=comm_ref.at[0],
                dst_ref=out_ref.at[pl.ds(s * m_per, m_per)],
                send_sem=send_sems.at[0],
                recv_sem=recv_sems.at[s],
                device_id=(s,),
                device_id_type=pl.DeviceIdType.MESH,
            )
            recv.wait_recv()

        for rdma in sends:
            rdma.wait_send()

    out_shape = jax.ShapeDtypeStruct((N_DEV * m_per, n_per), jnp.float32)
    return pl.pallas_call(
        body,
        out_shape=out_shape,
        in_specs=[
            pl.BlockSpec(memory_space=pltpu.VMEM),
            pl.BlockSpec(memory_space=pltpu.VMEM),
        ],
        out_specs=pl.BlockSpec(memory_space=pltpu.VMEM),
        scratch_shapes=[
            pltpu.VMEM((N_DEV, m_per, n_per), jnp.float32),
            pltpu.SemaphoreType.DMA((N_DEV,)),
            pltpu.SemaphoreType.DMA((N_DEV,)),
        ],
        compiler_params=pltpu.CompilerParams(collective_id=0),
    )(x, w_mat)


# device time: 5180 ns/iter; 2.6527x vs baseline; 2.6527x over previous
import os

import jax
import jax.numpy as jnp
from jax import lax
from jax.experimental import pallas as pl
from jax.experimental.pallas import tpu as pltpu

N_DEV = 8
try:
    _ABL = (os.path.join(os.path.dirname(__file__), "kabl.txt") and
            open(os.path.join(os.path.dirname(__file__), "kabl.txt")).read().strip())
except OSError:
    _ABL = ""


def kernel(x, w_mat):
    m_per, k = x.shape
    n = w_mat.shape[1]
    n_per = n // N_DEV

    if _ABL == "compute":
        def abl_body(x_ref, w_ref, out_ref, comm_ref, send_sems, recv_sems):
            x_blk = x_ref[:, :]
            for o in range(N_DEV):
                comm_ref[o, :, :] = jnp.dot(
                    x_blk,
                    w_ref[:, pl.ds(o * n_per, n_per)],
                    preferred_element_type=jnp.float32,
                )
                out_ref[pl.ds(o * m_per, m_per), :] = comm_ref[o, :, :]
        body_fn = abl_body
        return pl.pallas_call(
            body_fn,
            out_shape=jax.ShapeDtypeStruct((N_DEV * m_per, n_per), jnp.float32),
            in_specs=[
                pl.BlockSpec(memory_space=pltpu.VMEM),
                pl.BlockSpec(memory_space=pltpu.VMEM),
            ],
            out_specs=pl.BlockSpec(memory_space=pltpu.VMEM),
            scratch_shapes=[
                pltpu.VMEM((N_DEV, m_per, n_per), jnp.float32),
                pltpu.SemaphoreType.DMA((N_DEV,)),
                pltpu.SemaphoreType.DMA((N_DEV,)),
            ],
        )(x, w_mat)

    if _ABL == "biggemm":
        def abl_body(x_ref, w_ref, out_ref, y_ref, send_sems, recv_sems):
            y_ref[:, :] = jnp.dot(
                x_ref[:, :], w_ref[:, :], preferred_element_type=jnp.float32
            )
            for o in range(N_DEV):
                out_ref[pl.ds(o * m_per, m_per), :] = y_ref[
                    :, o * n_per:(o + 1) * n_per
                ]
        return pl.pallas_call(
            abl_body,
            out_shape=jax.ShapeDtypeStruct((N_DEV * m_per, n_per), jnp.float32),
            in_specs=[
                pl.BlockSpec(memory_space=pltpu.VMEM),
                pl.BlockSpec(memory_space=pltpu.VMEM),
            ],
            out_specs=pl.BlockSpec(memory_space=pltpu.VMEM),
            scratch_shapes=[
                pltpu.VMEM((m_per, n), jnp.float32),
                pltpu.SemaphoreType.DMA((N_DEV,)),
                pltpu.SemaphoreType.DMA((N_DEV,)),
            ],
        )(x, w_mat)

    def body(x_ref, w_ref, out_ref, comm_ref, send_sems, recv_sems):
        my = lax.axis_index("i")

        with jax.named_scope("barrier"):
            barrier_sem = pltpu.get_barrier_semaphore()
            for o in range(1, N_DEV):
                peer = lax.rem(my + o, N_DEV)
                pl.semaphore_signal(
                    barrier_sem, inc=1,
                    device_id=(peer,), device_id_type=pl.DeviceIdType.MESH,
                )
            pl.semaphore_wait(barrier_sem, N_DEV - 1)

        x_blk = x_ref[:, :]
        sends = []
        for o in range(1, N_DEV):
            with jax.named_scope(f"compute_send#o={o}"):
                j = lax.rem(my + o, N_DEV)
                comm_ref[o, :, :] = jnp.dot(
                    x_blk,
                    w_ref[:, pl.ds(j * n_per, n_per)],
                    preferred_element_type=jnp.float32,
                )
                rdma = pltpu.make_async_remote_copy(
                    src_ref=comm_ref.at[o],
                    dst_ref=out_ref.at[pl.ds(my * m_per, m_per)],
                    send_sem=send_sems.at[o],
                    recv_sem=recv_sems.at[my],
                    device_id=(j,),
                    device_id_type=pl.DeviceIdType.MESH,
                )
                rdma.start()
                sends.append(rdma)

        with jax.named_scope("own_block"):
            out_ref[pl.ds(my * m_per, m_per), :] = jnp.dot(
                x_blk,
                w_ref[:, pl.ds(my * n_per, n_per)],
                preferred_element_type=jnp.float32,
            )

        for o in range(1, N_DEV):
            with jax.named_scope(f"wait_recv#o={o}"):
                s = lax.rem(my + N_DEV - o, N_DEV)
                recv = pltpu.make_async_remote_copy(
                    src_ref=comm_ref.at[0],
                    dst_ref=out_ref.at[pl.ds(s * m_per, m_per)],
                    send_sem=send_sems.at[0],
                    recv_sem=recv_sems.at[s],
                    device_id=(s,),
                    device_id_type=pl.DeviceIdType.MESH,
                )
                recv.wait_recv()

        with jax.named_scope("drain_sends"):
            for rdma in sends:
                rdma.wait_send()

    out_shape = jax.ShapeDtypeStruct((N_DEV * m_per, n_per), jnp.float32)
    return pl.pallas_call(
        body,
        out_shape=out_shape,
        in_specs=[
            pl.BlockSpec(memory_space=pltpu.VMEM),
            pl.BlockSpec(memory_space=pltpu.VMEM),
        ],
        out_specs=pl.BlockSpec(memory_space=pltpu.VMEM),
        scratch_shapes=[
            pltpu.VMEM((N_DEV, m_per, n_per), jnp.float32),
            pltpu.SemaphoreType.DMA((N_DEV,)),
            pltpu.SemaphoreType.DMA((N_DEV,)),
        ],
        compiler_params=pltpu.CompilerParams(collective_id=0),
    )(x, w_mat)
